# Initial kernel scaffold; baseline (speedup 1.0000x reference)
import functools

import jax
import jax.numpy as jnp
from jax import lax
from jax.experimental import pallas as pl
from jax.experimental.pallas import tpu as pltpu
from jax.experimental.pallas import tpu_sc as plsc

N_NODES = 10000
IN_FEATS = 128
NUM_HEADS = 8
OUT_FEATS = 16
N_EDGES = 320000
NEG_SLOPE = 0.2

ROW = 144
CH = 128
NCH = N_EDGES // CH
NTILES = 32
ROWS_PER_TILE = N_NODES // 16
ZR = 25


def _proj_body(x_ref, wt_ref, alr_ref, featel_ref, er16_ref):
    f = jnp.dot(x_ref[...], wt_ref[...], preferred_element_type=jnp.float32)
    g = jnp.dot(f, alr_ref[...], preferred_element_type=jnp.float32)
    featel_ref[...] = jnp.concatenate([f, g[:, :16]], axis=1)
    er16_ref[...] = jnp.concatenate(
        [g[:, 8:16], jnp.zeros((f.shape[0], 8), jnp.float32)], axis=1)


def _finish_body(acc_ref, bias_ref, out_ref):
    u = acc_ref[0] + acc_ref[1]
    s = u[:, 128:136]
    r = jnp.where(s != 0.0, 1.0 / s, 0.0)
    parts = [u[:, h * 16:(h + 1) * 16] * r[:, h:h + 1] for h in range(NUM_HEADS)]
    out_ref[...] = jnp.concatenate(parts, axis=1) + bias_ref[...]


def _edge_body(featel, er16, srcs, dsts, out,
               src_v, dst_v, gbuf, ebuf, obuf, wbuf, zbuf, acc, sem1, sem2):
    cid = lax.axis_index("c")
    sid = lax.axis_index("s")
    wid = sid * 2 + cid

    iota = lax.iota(jnp.int32, 16)
    pairsel = lax.shift_right_logical(iota, 3)
    hsel = jnp.bitwise_and(iota, 7)
    col_el = 128 + hsel
    idx_hi = jnp.minimum(iota + 8, 15)
    mask8 = iota < 8
    zv = jnp.zeros((16,), jnp.float32)

    for i in range(ZR):
        for j in range(ROW // 16):
            zbuf[i, pl.ds(j * 16, 16)] = zv
    row0 = sid * ROWS_PER_TILE
    for r in range(0, ROWS_PER_TILE, ZR):
        pltpu.sync_copy(zbuf, acc.at[pl.ds(row0 + r, ZR)])
    plsc.subcore_barrier()

    nch = jnp.where(wid < NCH % NTILES, NCH // NTILES + 1, NCH // NTILES)

    def chunk_body(k, carry):
        c = wid + k * NTILES
        base = c * CH
        pltpu.sync_copy(srcs.at[pl.ds(base, CH)], src_v)
        pltpu.sync_copy(dsts.at[pl.ds(base, CH)], dst_v)
        cp1 = pltpu.async_copy(featel.at[src_v], gbuf, sem1)
        cp2 = pltpu.async_copy(er16.at[dst_v], ebuf, sem2)
        cp1.wait()
        cp2.wait()

        def pair_body(p, inner):
            e0 = 2 * p
            rows = e0 + pairsel
            el2 = plsc.load_gather(gbuf, [rows, col_el])
            er2 = plsc.load_gather(ebuf, [rows, hsel])
            z = el2 + er2
            w = jnp.exp(jnp.maximum(z, z * NEG_SLOPE))
            wbuf[...] = w
            whi = plsc.load_gather(wbuf, [idx_hi])
            obuf[e0, pl.ds(128, 16)] = jnp.where(mask8, w, 0.0)
            obuf[e0 + 1, pl.ds(128, 16)] = jnp.where(mask8, whi, 0.0)
            for h in range(NUM_HEADS):
                off = h * 16
                obuf[e0, pl.ds(off, 16)] = gbuf[e0, pl.ds(off, 16)] * wbuf[h]
                obuf[e0 + 1, pl.ds(off, 16)] = (
                    gbuf[e0 + 1, pl.ds(off, 16)] * wbuf[8 + h])
            return inner

        lax.fori_loop(0, CH // 2, pair_body, 0)
        pltpu.sync_copy(obuf, acc.at[dst_v], add=True)
        return carry

    lax.fori_loop(0, nch, chunk_body, 0)
    plsc.subcore_barrier()

    pltpu.sync_copy(acc.at[pl.ds(row0, ROWS_PER_TILE)],
                    out.at[cid, pl.ds(row0, ROWS_PER_TILE)])


def kernel(x, edge_index, W, attn_l, attn_r, bias):
    src = edge_index[0].astype(jnp.int32)
    dst = edge_index[1].astype(jnp.int32)

    eye = jnp.eye(NUM_HEADS, dtype=jnp.float32)
    al = (eye[:, None, :] * attn_l[0][:, :, None]).reshape(IN_FEATS, NUM_HEADS)
    ar = (eye[:, None, :] * attn_r[0][:, :, None]).reshape(IN_FEATS, NUM_HEADS)
    alr = jnp.zeros((IN_FEATS, IN_FEATS), jnp.float32)
    alr = alr.at[:, :NUM_HEADS].set(al).at[:, NUM_HEADS:2 * NUM_HEADS].set(ar)

    blk = 1000
    grid = N_NODES // blk
    featel, er16 = pl.pallas_call(
        _proj_body,
        grid=(grid,),
        in_specs=[
            pl.BlockSpec((blk, IN_FEATS), lambda i: (i, 0)),
            pl.BlockSpec((IN_FEATS, IN_FEATS), lambda i: (0, 0)),
            pl.BlockSpec((IN_FEATS, IN_FEATS), lambda i: (0, 0)),
        ],
        out_specs=[
            pl.BlockSpec((blk, ROW), lambda i: (i, 0)),
            pl.BlockSpec((blk, 16), lambda i: (i, 0)),
        ],
        out_shape=[
            jax.ShapeDtypeStruct((N_NODES, ROW), jnp.float32),
            jax.ShapeDtypeStruct((N_NODES, 16), jnp.float32),
        ],
    )(x, W.T, alr)

    edge_kernel = functools.partial(
        pl.kernel,
        out_type=jax.ShapeDtypeStruct((2, N_NODES, ROW), jnp.float32),
        mesh=plsc.VectorSubcoreMesh(core_axis_name="c", subcore_axis_name="s"),
        scratch_types=[
            pltpu.VMEM((CH,), jnp.int32),
            pltpu.VMEM((CH,), jnp.int32),
            pltpu.VMEM((CH, ROW), jnp.float32),
            pltpu.VMEM((CH, 16), jnp.float32),
            pltpu.VMEM((CH, ROW), jnp.float32),
            pltpu.VMEM((16,), jnp.float32),
            pltpu.VMEM((ZR, ROW), jnp.float32),
            pltpu.VMEM_SHARED((N_NODES, ROW), jnp.float32),
            pltpu.SemaphoreType.DMA,
            pltpu.SemaphoreType.DMA,
        ],
    )(_edge_body)
    acc2 = edge_kernel(featel, er16, src, dst)

    out = pl.pallas_call(
        _finish_body,
        grid=(grid,),
        in_specs=[
            pl.BlockSpec((2, blk, ROW), lambda i: (0, i, 0)),
            pl.BlockSpec((1, IN_FEATS), lambda i: (0, 0)),
        ],
        out_specs=pl.BlockSpec((blk, IN_FEATS), lambda i: (i, 0)),
        out_shape=jax.ShapeDtypeStruct((N_NODES, IN_FEATS), jnp.float32),
    )(acc2, bias.reshape(1, IN_FEATS))

    return out.reshape(N_NODES, NUM_HEADS, OUT_FEATS)


# speedup vs baseline: 60.8267x; 60.8267x over previous
import functools

import jax
import jax.numpy as jnp
from jax import lax
from jax.experimental import pallas as pl
from jax.experimental.pallas import tpu as pltpu
from jax.experimental.pallas import tpu_sc as plsc

N_NODES = 10000
IN_FEATS = 128
NUM_HEADS = 8
OUT_FEATS = 16
N_EDGES = 320000
NEG_SLOPE = 0.2

ROW = 144
CH = 64
NCH = N_EDGES // CH
NTILES = 32
N_PAD = 10240
ROWS_PER_TILE = N_PAD // 16
ZR = 32


def _proj_body(x_ref, wt_ref, alr_ref, featel_ref, er16_ref):
    f = jnp.dot(x_ref[...], wt_ref[...], preferred_element_type=jnp.float32)
    g = jnp.dot(f, alr_ref[...], preferred_element_type=jnp.float32)
    featel_ref[...] = jnp.concatenate([f, g[:, :16]], axis=1)
    er16_ref[...] = jnp.concatenate(
        [g[:, 8:16], jnp.zeros((f.shape[0], 8), jnp.float32)], axis=1)


def _finish_body(acc_ref, bias_ref, out_ref):
    u = acc_ref[0] + acc_ref[1]
    s = u[:, 128:136]
    r = jnp.where(s != 0.0, 1.0 / s, 0.0)
    parts = [u[:, h * 16:(h + 1) * 16] * r[:, h:h + 1] for h in range(NUM_HEADS)]
    out_ref[...] = jnp.concatenate(parts, axis=1) + bias_ref[...]


def _edge_body(featel, er16, srcs, dsts, out,
               src_v, dst_v, gbuf, ebuf, obuf, zbuf, acc, sem1, sem2):
    cid = lax.axis_index("c")
    sid = lax.axis_index("s")
    wid = sid * 2 + cid

    iota = lax.iota(jnp.int32, 16)
    mask8 = iota < 8
    zv = jnp.zeros((16,), jnp.float32)

    for i in range(ZR):
        for j in range(ROW // 16):
            zbuf[i, pl.ds(j * 16, 16)] = zv
    row0 = sid * ROWS_PER_TILE
    for r in range(0, ROWS_PER_TILE, ZR):
        pltpu.sync_copy(zbuf, acc.at[pl.ds(row0 + r, ZR)])
    plsc.subcore_barrier()

    nch = jnp.where(wid < NCH % NTILES, NCH // NTILES + 1, NCH // NTILES)

    def chunk_body(k, carry):
        c = wid + k * NTILES
        base = c * CH
        pltpu.sync_copy(srcs.at[pl.ds(base, CH)], src_v)
        pltpu.sync_copy(dsts.at[pl.ds(base, CH)], dst_v)
        cp1 = pltpu.async_copy(featel.at[src_v], gbuf, sem1)
        cp2 = pltpu.async_copy(er16.at[dst_v], ebuf, sem2)
        cp1.wait()
        cp2.wait()

        def edge_iter(e, inner):
            el16 = gbuf[e, pl.ds(128, 16)]
            er16v = ebuf[e, pl.ds(0, 16)]
            z = el16 + er16v
            w = jnp.exp(jnp.maximum(z, z * NEG_SLOPE))
            obuf[e, pl.ds(128, 16)] = jnp.where(mask8, w, 0.0)
            for h in range(NUM_HEADS):
                off = h * 16
                obuf[e, pl.ds(off, 16)] = gbuf[e, pl.ds(off, 16)] * w[h]
            return inner

        lax.fori_loop(0, CH, edge_iter, 0)
        pltpu.sync_copy(obuf, acc.at[dst_v], add=True)
        return carry

    lax.fori_loop(0, nch, chunk_body, 0)
    plsc.subcore_barrier()

    pltpu.sync_copy(acc.at[pl.ds(row0, ROWS_PER_TILE)],
                    out.at[cid, pl.ds(row0, ROWS_PER_TILE)])


def kernel(x, edge_index, W, attn_l, attn_r, bias):
    src = edge_index[0].astype(jnp.int32)
    dst = edge_index[1].astype(jnp.int32)

    eye = jnp.eye(NUM_HEADS, dtype=jnp.float32)
    al = (eye[:, None, :] * attn_l[0][:, :, None]).reshape(IN_FEATS, NUM_HEADS)
    ar = (eye[:, None, :] * attn_r[0][:, :, None]).reshape(IN_FEATS, NUM_HEADS)
    alr = jnp.zeros((IN_FEATS, IN_FEATS), jnp.float32)
    alr = alr.at[:, :NUM_HEADS].set(al).at[:, NUM_HEADS:2 * NUM_HEADS].set(ar)

    blk = 1000
    grid = N_NODES // blk
    featel, er16 = pl.pallas_call(
        _proj_body,
        grid=(grid,),
        in_specs=[
            pl.BlockSpec((blk, IN_FEATS), lambda i: (i, 0)),
            pl.BlockSpec((IN_FEATS, IN_FEATS), lambda i: (0, 0)),
            pl.BlockSpec((IN_FEATS, IN_FEATS), lambda i: (0, 0)),
        ],
        out_specs=[
            pl.BlockSpec((blk, ROW), lambda i: (i, 0)),
            pl.BlockSpec((blk, 16), lambda i: (i, 0)),
        ],
        out_shape=[
            jax.ShapeDtypeStruct((N_NODES, ROW), jnp.float32),
            jax.ShapeDtypeStruct((N_NODES, 16), jnp.float32),
        ],
    )(x, W.T, alr)

    edge_kernel = functools.partial(
        pl.kernel,
        out_type=jax.ShapeDtypeStruct((2, N_PAD, ROW), jnp.float32),
        mesh=plsc.VectorSubcoreMesh(core_axis_name="c", subcore_axis_name="s"),
        compiler_params=pltpu.CompilerParams(use_tc_tiling_on_sc=False),
        scratch_types=[
            pltpu.VMEM((CH,), jnp.int32),
            pltpu.VMEM((CH,), jnp.int32),
            pltpu.VMEM((CH, ROW), jnp.float32),
            pltpu.VMEM((CH, 16), jnp.float32),
            pltpu.VMEM((CH, ROW), jnp.float32),
            pltpu.VMEM((ZR, ROW), jnp.float32),
            pltpu.VMEM_SHARED((N_PAD, ROW), jnp.float32),
            pltpu.SemaphoreType.DMA,
            pltpu.SemaphoreType.DMA,
        ],
    )(_edge_body)
    acc2 = edge_kernel(featel, er16, src, dst)

    out = pl.pallas_call(
        _finish_body,
        grid=(grid,),
        in_specs=[
            pl.BlockSpec((2, blk, ROW), lambda i: (0, i, 0)),
            pl.BlockSpec((1, IN_FEATS), lambda i: (0, 0)),
        ],
        out_specs=pl.BlockSpec((blk, IN_FEATS), lambda i: (i, 0)),
        out_shape=jax.ShapeDtypeStruct((N_NODES, IN_FEATS), jnp.float32),
    )(acc2, bias.reshape(1, IN_FEATS))

    return out.reshape(N_NODES, NUM_HEADS, OUT_FEATS)


# speedup vs baseline: 123.8132x; 2.0355x over previous
import functools

import jax
import jax.numpy as jnp
from jax import lax
from jax.experimental import pallas as pl
from jax.experimental.pallas import tpu as pltpu
from jax.experimental.pallas import tpu_sc as plsc

N_NODES = 10000
IN_FEATS = 128
NUM_HEADS = 8
OUT_FEATS = 16
N_EDGES = 320000
NEG_SLOPE = 0.2

ROW = 144
CH = 64
NCH = N_EDGES // CH
NTILES = 32
NFULL = NCH // NTILES
NEXTRA = NCH % NTILES
N_PAD = 10112
ROWS_PER_TILE = N_PAD // 16


def _proj_body(x_ref, wt_ref, alr_ref, featel_ref, er16_ref):
    f = jnp.dot(x_ref[...], wt_ref[...], preferred_element_type=jnp.float32)
    g = jnp.dot(f, alr_ref[...], preferred_element_type=jnp.float32)
    featel_ref[...] = jnp.concatenate([f, g[:, :16]], axis=1)
    er16_ref[...] = jnp.concatenate(
        [g[:, 8:16], jnp.zeros((f.shape[0], 8), jnp.float32)], axis=1)


def _finish_body(acc_ref, bias_ref, out_ref):
    u = acc_ref[0] + acc_ref[1]
    s = u[:, 128:136]
    r = jnp.where(s != 0.0, 1.0 / s, 0.0)
    parts = [u[:, h * 16:(h + 1) * 16] * r[:, h:h + 1] for h in range(NUM_HEADS)]
    out_ref[...] = jnp.concatenate(parts, axis=1) + bias_ref[...]


def _edge_body(featel, er16, idxc, out,
               ibuf0, ibuf1, srcv0, srcv1, dstv0, dstv1,
               gbuf0, gbuf1, ebuf0, ebuf1, obuf0, obuf1, acc,
               isem0, isem1, gsem0, gsem1, esem0, esem1, ssem0, ssem1):
    cid = lax.axis_index("c")
    sid = lax.axis_index("s")
    wid = sid * 2 + cid

    ibuf = (ibuf0, ibuf1)
    srcv = (srcv0, srcv1)
    dstv = (dstv0, dstv1)
    gbuf = (gbuf0, gbuf1)
    ebuf = (ebuf0, ebuf1)
    obuf = (obuf0, obuf1)
    isem = (isem0, isem1)
    gsem = (gsem0, gsem1)
    esem = (esem0, esem1)
    ssem = (ssem0, ssem1)

    iota = lax.iota(jnp.int32, 16)
    mask8 = iota < 8
    zv = jnp.zeros((16,), jnp.float32)
    has_extra = wid < NEXTRA

    for i in range(CH):
        for j in range(ROW // 16):
            obuf0[i, pl.ds(j * 16, 16)] = zv
    row0 = sid * ROWS_PER_TILE
    nzfull = (ROWS_PER_TILE // CH) * CH
    for r in range(0, nzfull, CH):
        pltpu.sync_copy(obuf0, acc.at[pl.ds(row0 + r, CH)])
    pltpu.sync_copy(obuf0.at[pl.ds(0, ROWS_PER_TILE - nzfull)],
                    acc.at[pl.ds(row0 + nzfull, ROWS_PER_TILE - nzfull)])
    plsc.subcore_barrier()

    def chunk_of(j):
        return wid + j * NTILES

    def build_src(b):
        for i in range(CH // 16):
            srcv[b][pl.ds(i * 16, 16)] = ibuf[b][pl.ds(i * 16, 16)]

    def build_dst(b):
        for i in range(CH // 16):
            dstv[b][pl.ds(i * 16, 16)] = ibuf[b][pl.ds(CH + i * 16, 16)]

    def start_gathers(b):
        pltpu.async_copy(featel.at[srcv[b]], gbuf[b], gsem[b])
        pltpu.async_copy(er16.at[ibuf[b].at[pl.ds(CH, CH)]], ebuf[b], esem[b])

    def wait_gathers(b):
        pltpu.make_async_copy(featel.at[srcv[b]], gbuf[b], gsem[b]).wait()
        pltpu.make_async_copy(er16.at[ibuf[b].at[pl.ds(CH, CH)]], ebuf[b],
                              esem[b]).wait()

    def wait_scatter(b):
        pltpu.make_async_copy(obuf[b], acc.at[dstv[b]], ssem[b]).wait()

    def compute_chunk(b):
        def edge_iter(e, inner):
            el16 = gbuf[b][e, pl.ds(128, 16)]
            er16v = ebuf[b][e, pl.ds(0, 16)]
            z = el16 + er16v
            w = jnp.exp(jnp.maximum(z, z * NEG_SLOPE))
            obuf[b][e, pl.ds(128, 16)] = jnp.where(mask8, w, 0.0)
            for h in range(NUM_HEADS):
                off = h * 16
                obuf[b][e, pl.ds(off, 16)] = gbuf[b][e, pl.ds(off, 16)] * w[h]
            return inner
        lax.fori_loop(0, CH, edge_iter, 0)

    pltpu.sync_copy(idxc.at[chunk_of(0)], ibuf[0])
    pltpu.async_copy(idxc.at[chunk_of(1)], ibuf[1], isem[1])
    build_src(0)
    start_gathers(0)

    def pair_body(p, carry):
        for b in range(2):
            j = 2 * p + b
            wait_gathers(b)

            @pl.when(p >= 1)
            def _():
                wait_scatter(b)
            build_dst(b)

            last_p = NFULL // 2 - 1
            next2_ok = ((p < last_p) | has_extra) if b == 0 else (p < last_p)

            @pl.when(next2_ok)
            def _():
                pltpu.async_copy(idxc.at[chunk_of(j + 2)], ibuf[b], isem[b])

            next1_ok = (p <= last_p) if b == 0 else ((p < last_p) | has_extra)

            @pl.when(next1_ok)
            def _():
                pltpu.make_async_copy(idxc.at[chunk_of(j + 1)], ibuf[1 - b],
                                      isem[1 - b]).wait()
                build_src(1 - b)
                start_gathers(1 - b)

            compute_chunk(b)
            pltpu.async_copy(obuf[b], acc.at[dstv[b]], ssem[b], add=True)
        return carry

    lax.fori_loop(0, NFULL // 2, pair_body, 0)

    @pl.when(has_extra)
    def _():
        wait_gathers(0)
        wait_scatter(0)
        build_dst(0)
        compute_chunk(0)
        pltpu.async_copy(obuf[0], acc.at[dstv[0]], ssem[0], add=True)

    wait_scatter(0)
    wait_scatter(1)
    plsc.subcore_barrier()

    pltpu.sync_copy(acc.at[pl.ds(row0, ROWS_PER_TILE)],
                    out.at[cid, pl.ds(row0, ROWS_PER_TILE)])


def kernel(x, edge_index, W, attn_l, attn_r, bias):
    src = edge_index[0].astype(jnp.int32)
    dst = edge_index[1].astype(jnp.int32)

    eye = jnp.eye(NUM_HEADS, dtype=jnp.float32)
    al = (eye[:, None, :] * attn_l[0][:, :, None]).reshape(IN_FEATS, NUM_HEADS)
    ar = (eye[:, None, :] * attn_r[0][:, :, None]).reshape(IN_FEATS, NUM_HEADS)
    alr = jnp.zeros((IN_FEATS, IN_FEATS), jnp.float32)
    alr = alr.at[:, :NUM_HEADS].set(al).at[:, NUM_HEADS:2 * NUM_HEADS].set(ar)

    blk = 1000
    grid = N_NODES // blk
    featel, er16 = pl.pallas_call(
        _proj_body,
        grid=(grid,),
        in_specs=[
            pl.BlockSpec((blk, IN_FEATS), lambda i: (i, 0)),
            pl.BlockSpec((IN_FEATS, IN_FEATS), lambda i: (0, 0)),
            pl.BlockSpec((IN_FEATS, IN_FEATS), lambda i: (0, 0)),
        ],
        out_specs=[
            pl.BlockSpec((blk, ROW), lambda i: (i, 0)),
            pl.BlockSpec((blk, 16), lambda i: (i, 0)),
        ],
        out_shape=[
            jax.ShapeDtypeStruct((N_NODES, ROW), jnp.float32),
            jax.ShapeDtypeStruct((N_NODES, 16), jnp.float32),
        ],
    )(x, W.T, alr)

    idxc = jnp.concatenate(
        [src.reshape(NCH, CH), dst.reshape(NCH, CH)], axis=1)

    edge_kernel = functools.partial(
        pl.kernel,
        out_type=jax.ShapeDtypeStruct((2, N_PAD, ROW), jnp.float32),
        mesh=plsc.VectorSubcoreMesh(core_axis_name="c", subcore_axis_name="s"),
        compiler_params=pltpu.CompilerParams(use_tc_tiling_on_sc=False),
        scratch_types=[
            pltpu.VMEM((2 * CH,), jnp.int32),
            pltpu.VMEM((2 * CH,), jnp.int32),
            pltpu.VMEM((CH,), jnp.int32),
            pltpu.VMEM((CH,), jnp.int32),
            pltpu.VMEM((CH,), jnp.int32),
            pltpu.VMEM((CH,), jnp.int32),
            pltpu.VMEM((CH, ROW), jnp.float32),
            pltpu.VMEM((CH, ROW), jnp.float32),
            pltpu.VMEM((CH, 16), jnp.float32),
            pltpu.VMEM((CH, 16), jnp.float32),
            pltpu.VMEM((CH, ROW), jnp.float32),
            pltpu.VMEM((CH, ROW), jnp.float32),
            pltpu.VMEM_SHARED((N_PAD, ROW), jnp.float32),
        ] + [pltpu.SemaphoreType.DMA] * 8,
    )(_edge_body)
    acc2 = edge_kernel(featel, er16, idxc)

    out = pl.pallas_call(
        _finish_body,
        grid=(grid,),
        in_specs=[
            pl.BlockSpec((2, blk, ROW), lambda i: (0, i, 0)),
            pl.BlockSpec((1, IN_FEATS), lambda i: (0, 0)),
        ],
        out_specs=pl.BlockSpec((blk, IN_FEATS), lambda i: (i, 0)),
        out_shape=jax.ShapeDtypeStruct((N_NODES, IN_FEATS), jnp.float32),
    )(acc2, bias.reshape(1, IN_FEATS))

    return out.reshape(N_NODES, NUM_HEADS, OUT_FEATS)
